# speedup vs baseline: 1.3129x; 1.3129x over previous
import jax
import jax.numpy as jnp
from jax.experimental import pallas as pl

N_TOK = 32768
D = 768
NUM_OUTPUTS = 10
BLK = 1024


def _fused_body(x_ref, w_ref, b_ref, out_ref, y_ref, conf_ref):
    x = x_ref[...]
    z = jnp.dot(x, w_ref[...], preferred_element_type=jnp.float32)
    z = z + b_ref[...]
    v = z[:, 0:1]
    conf = jax.nn.sigmoid(v)
    mask = conf > 0.5
    conf_ref[...] = conf
    y_ref[...] = jnp.where(mask, z[:, 1:], 0.0)
    out_ref[...] = jnp.where(mask, 0.0, x)


def kernel(X, Wg, bg, Wc, bc):
    w_all = jnp.concatenate([Wg, Wc], axis=1)
    b_all = jnp.concatenate([bg, bc]).reshape(1, 1 + NUM_OUTPUTS)

    grid = (N_TOK // BLK,)
    out, y_hat, conf = pl.pallas_call(
        _fused_body,
        grid=grid,
        in_specs=[
            pl.BlockSpec((BLK, D), lambda i: (i, 0)),
            pl.BlockSpec((D, 1 + NUM_OUTPUTS), lambda i: (0, 0)),
            pl.BlockSpec((1, 1 + NUM_OUTPUTS), lambda i: (0, 0)),
        ],
        out_specs=[
            pl.BlockSpec((BLK, D), lambda i: (i, 0)),
            pl.BlockSpec((BLK, NUM_OUTPUTS), lambda i: (i, 0)),
            pl.BlockSpec((BLK, 1), lambda i: (i, 0)),
        ],
        out_shape=[
            jax.ShapeDtypeStruct((N_TOK, D), jnp.float32),
            jax.ShapeDtypeStruct((N_TOK, NUM_OUTPUTS), jnp.float32),
            jax.ShapeDtypeStruct((N_TOK, 1), jnp.float32),
        ],
    )(X, w_all, b_all)
    return out, y_hat, conf.reshape(-1)


# speedup vs baseline: 1.3501x; 1.0283x over previous
import jax
import jax.numpy as jnp
from jax.experimental import pallas as pl

N_TOK = 32768
D = 768
NUM_OUTPUTS = 10
BLK = 2048


def _fused_body(x_ref, w_ref, b_ref, out_ref, y_ref, conf_ref):
    x = x_ref[...]
    z = jnp.dot(x, w_ref[...], preferred_element_type=jnp.float32)
    z = z + b_ref[...]
    v = z[:, 0:1]
    conf = jax.nn.sigmoid(v)
    mask = conf > 0.5
    conf_ref[...] = conf
    y_ref[...] = jnp.where(mask, z[:, 1:], 0.0)
    out_ref[...] = jnp.where(mask, 0.0, x)


def kernel(X, Wg, bg, Wc, bc):
    w_all = jnp.concatenate([Wg, Wc], axis=1)
    b_all = jnp.concatenate([bg, bc]).reshape(1, 1 + NUM_OUTPUTS)

    grid = (N_TOK // BLK,)
    out, y_hat, conf = pl.pallas_call(
        _fused_body,
        grid=grid,
        in_specs=[
            pl.BlockSpec((BLK, D), lambda i: (i, 0)),
            pl.BlockSpec((D, 1 + NUM_OUTPUTS), lambda i: (0, 0)),
            pl.BlockSpec((1, 1 + NUM_OUTPUTS), lambda i: (0, 0)),
        ],
        out_specs=[
            pl.BlockSpec((BLK, D), lambda i: (i, 0)),
            pl.BlockSpec((BLK, NUM_OUTPUTS), lambda i: (i, 0)),
            pl.BlockSpec((BLK, 1), lambda i: (i, 0)),
        ],
        out_shape=[
            jax.ShapeDtypeStruct((N_TOK, D), jnp.float32),
            jax.ShapeDtypeStruct((N_TOK, NUM_OUTPUTS), jnp.float32),
            jax.ShapeDtypeStruct((N_TOK, 1), jnp.float32),
        ],
    )(X, w_all, b_all)
    return out, y_hat, conf.reshape(-1)


# speedup vs baseline: 1.3993x; 1.0364x over previous
import jax
import jax.numpy as jnp
from jax.experimental import pallas as pl

N_TOK = 32768
D = 768
NUM_OUTPUTS = 10
BLK = 4096


def _fused_body(x_ref, w_ref, b_ref, out_ref, y_ref, conf_ref):
    x = x_ref[...]
    z = jnp.dot(x, w_ref[...], preferred_element_type=jnp.float32)
    z = z + b_ref[...]
    v = z[:, 0:1]
    conf = jax.nn.sigmoid(v)
    mask = conf > 0.5
    conf_ref[...] = conf
    y_ref[...] = jnp.where(mask, z[:, 1:], 0.0)
    out_ref[...] = jnp.where(mask, 0.0, x)


def kernel(X, Wg, bg, Wc, bc):
    w_all = jnp.concatenate([Wg, Wc], axis=1)
    b_all = jnp.concatenate([bg, bc]).reshape(1, 1 + NUM_OUTPUTS)

    grid = (N_TOK // BLK,)
    out, y_hat, conf = pl.pallas_call(
        _fused_body,
        grid=grid,
        in_specs=[
            pl.BlockSpec((BLK, D), lambda i: (i, 0)),
            pl.BlockSpec((D, 1 + NUM_OUTPUTS), lambda i: (0, 0)),
            pl.BlockSpec((1, 1 + NUM_OUTPUTS), lambda i: (0, 0)),
        ],
        out_specs=[
            pl.BlockSpec((BLK, D), lambda i: (i, 0)),
            pl.BlockSpec((BLK, NUM_OUTPUTS), lambda i: (i, 0)),
            pl.BlockSpec((BLK, 1), lambda i: (i, 0)),
        ],
        out_shape=[
            jax.ShapeDtypeStruct((N_TOK, D), jnp.float32),
            jax.ShapeDtypeStruct((N_TOK, NUM_OUTPUTS), jnp.float32),
            jax.ShapeDtypeStruct((N_TOK, 1), jnp.float32),
        ],
    )(X, w_all, b_all)
    return out, y_hat, conf.reshape(-1)
